# baseline (speedup 1.0000x reference)
import functools

import jax
import jax.numpy as jnp
from jax import lax
from jax.experimental import pallas as pl
from jax.experimental.pallas import tpu as pltpu
from jax.experimental.pallas import tpu_sc as plsc

ENT_NUM = 100000
DIM = 64
B = 16384
GAMMA = 12.0

_NC = 2
_NS = 16
_NW = _NC * _NS
_BPW = B // _NW
_CHUNK = 256
_NCHUNK = _BPW // _CHUNK
_CBLK = 16384


def _entcat_body(a_ref, b_ref, o_ref):
    ab = jnp.concatenate([a_ref[...], b_ref[...]], axis=0)
    o_ref[...] = ab.T


def _entcat(a, b):
    spec = pl.BlockSpec((DIM, _CBLK), lambda i: (0, i))
    return pl.pallas_call(
        _entcat_body,
        grid=(pl.cdiv(ENT_NUM, _CBLK),),
        in_specs=[spec, spec],
        out_specs=pl.BlockSpec((_CBLK, 2 * DIM), lambda i: (i, 0)),
        out_shape=jax.ShapeDtypeStruct((ENT_NUM, 2 * DIM), jnp.float32),
    )(a, b)


def _sc_gather2(table, idx0, idx1):
    mesh = plsc.VectorSubcoreMesh(core_axis_name="c", subcore_axis_name="s")
    out_t = [jax.ShapeDtypeStruct((B, 2 * DIM), jnp.float32) for _ in range(2)]

    @functools.partial(
        pl.kernel,
        mesh=mesh,
        out_type=out_t,
        scratch_types=[
            pltpu.VMEM((_BPW,), jnp.int32),
            pltpu.VMEM((_BPW,), jnp.int32),
            pltpu.VMEM((_CHUNK, 2 * DIM), jnp.float32),
            pltpu.VMEM((_CHUNK, 2 * DIM), jnp.float32),
            pltpu.SemaphoreType.DMA,
        ],
    )
    def k(t_hbm, i0_hbm, i1_hbm, o0, o1, i0_v, i1_v, b0, b1, sem):
        wid = lax.axis_index("s") * _NC + lax.axis_index("c")
        base = wid * _BPW
        pltpu.sync_copy(i0_hbm.at[pl.ds(base, _BPW)], i0_v)
        pltpu.sync_copy(i1_hbm.at[pl.ds(base, _BPW)], i1_v)

        def body(c, _):
            off = c * _CHUNK
            cp0 = pltpu.async_copy(t_hbm.at[i0_v.at[pl.ds(off, _CHUNK)]], b0, sem)
            cp1 = pltpu.async_copy(t_hbm.at[i1_v.at[pl.ds(off, _CHUNK)]], b1, sem)
            cp0.wait()
            cp1.wait()
            dst = pl.ds(base + off, _CHUNK)
            pltpu.sync_copy(b0, o0.at[dst])
            pltpu.sync_copy(b1, o1.at[dst])

        lax.fori_loop(0, _NCHUNK, body, None, unroll=False)

    return k(table, idx0, idx1)


def _sc_gather1(table, idx0, width):
    mesh = plsc.VectorSubcoreMesh(core_axis_name="c", subcore_axis_name="s")
    out_t = jax.ShapeDtypeStruct((B, width), jnp.float32)

    @functools.partial(
        pl.kernel,
        mesh=mesh,
        out_type=out_t,
        scratch_types=[
            pltpu.VMEM((_BPW,), jnp.int32),
            pltpu.VMEM((_CHUNK, width), jnp.float32),
            pltpu.SemaphoreType.DMA,
        ],
    )
    def k(t_hbm, i0_hbm, o0, i0_v, b0, sem):
        wid = lax.axis_index("s") * _NC + lax.axis_index("c")
        base = wid * _BPW
        pltpu.sync_copy(i0_hbm.at[pl.ds(base, _BPW)], i0_v)

        def body(c, _):
            off = c * _CHUNK
            cp0 = pltpu.async_copy(t_hbm.at[i0_v.at[pl.ds(off, _CHUNK)]], b0, sem)
            cp0.wait()
            pltpu.sync_copy(b0, o0.at[pl.ds(base + off, _CHUNK)])

        lax.fori_loop(0, _NCHUNK, body, None, unroll=False)

    return k(table, idx0)


def _score_body(h_ref, t_ref, rw_ref, o_ref):
    blk = h_ref.shape[0]
    hh = h_ref[...]
    tt = t_ref[...]
    rw = rw_ref[...]
    lane = lax.broadcasted_iota(jnp.int32, (blk, 2 * DIM), 1)
    re_half = lane < DIM
    rw_swap = pltpu.roll(rw, DIM, 1)
    w2 = jnp.where(re_half, rw_swap, rw)
    cs = jnp.where(re_half, jnp.cos(rw),
                   pltpu.roll(jnp.sin(rw), DIM, 1))

    row = lax.broadcasted_iota(jnp.int32, (2 * DIM, 2 * DIM), 0)
    col = lax.broadcasted_iota(jnp.int32, (2 * DIM, 2 * DIM), 1)
    m = ((row < DIM) == (col < DIM)).astype(jnp.float32)

    def mxu(x, y):
        return jax.lax.dot(x, y, preferred_element_type=jnp.float32)

    d_h = mxu(w2 * hh, m)
    d_t = mxu(w2 * tt, m)
    p = hh - d_h * w2
    pt = tt - d_t * w2

    swap_cs = pltpu.roll(cs, DIM, 1)
    cc = jnp.where(re_half, cs, swap_cs)
    ssn = jnp.where(re_half, -swap_cs, cs)
    swap_p = pltpu.roll(p, DIM, 1)
    s = p * cc + swap_p * ssn - pt
    sq = s * s
    mag = jnp.sqrt(sq + pltpu.roll(sq, DIM, 1))
    ones = jnp.ones((2 * DIM, 1), jnp.float32)
    o_ref[...] = 0.5 * mxu(mag, ones) - GAMMA


def _tc_score(h, t, rw):
    blk = 4096
    return pl.pallas_call(
        _score_body,
        grid=(B // blk,),
        in_specs=[
            pl.BlockSpec((blk, 2 * DIM), lambda i: (i, 0)),
            pl.BlockSpec((blk, 2 * DIM), lambda i: (i, 0)),
            pl.BlockSpec((blk, 2 * DIM), lambda i: (i, 0)),
        ],
        out_specs=pl.BlockSpec((blk, 1), lambda i: (i, 0)),
        out_shape=jax.ShapeDtypeStruct((B, 1), jnp.float32),
    )(h, t, rw)


def kernel(pos_sample, ent_embd, ent_embd_im, rel_embd, wr):
    h_idx = pos_sample[:, 0]
    r_idx = pos_sample[:, 1]
    t_idx = pos_sample[:, 2]
    entcat = _entcat(ent_embd.T, ent_embd_im.T)
    h, t = _sc_gather2(entcat, h_idx, t_idx)
    relcat = _entcat(rel_embd.T, wr.T)
    r = _sc_gather1(relcat, r_idx, 2 * DIM)
    return _tc_score(h, t, r)


# speedup vs baseline: 1.0034x; 1.0034x over previous
import functools

import jax
import jax.numpy as jnp
from jax import lax
from jax.experimental import pallas as pl
from jax.experimental.pallas import tpu as pltpu
from jax.experimental.pallas import tpu_sc as plsc

ENT_NUM = 100000
DIM = 64
B = 16384
GAMMA = 12.0

_NC = 2
_NS = 16
_NW = _NC * _NS
_BPW = B // _NW
_CHUNK = 256
_NCHUNK = _BPW // _CHUNK
_CBLK = 16384


def _entcat_body(a_ref, b_ref, o_ref):
    ab = jnp.concatenate([a_ref[...], b_ref[...]], axis=0)
    o_ref[...] = ab.T


def _entcat(a, b):
    spec = pl.BlockSpec((DIM, _CBLK), lambda i: (0, i))
    return pl.pallas_call(
        _entcat_body,
        grid=(pl.cdiv(ENT_NUM, _CBLK),),
        in_specs=[spec, spec],
        out_specs=pl.BlockSpec((_CBLK, 2 * DIM), lambda i: (i, 0)),
        out_shape=jax.ShapeDtypeStruct((ENT_NUM, 2 * DIM), jnp.float32),
    )(a, b)


def _sc_gather2(table, idx0, idx1):
    mesh = plsc.VectorSubcoreMesh(core_axis_name="c", subcore_axis_name="s")
    out_t = [jax.ShapeDtypeStruct((B, 2 * DIM), jnp.float32) for _ in range(2)]

    @functools.partial(
        pl.kernel,
        mesh=mesh,
        out_type=out_t,
        scratch_types=[
            pltpu.VMEM((_BPW,), jnp.int32),
            pltpu.VMEM((_BPW,), jnp.int32),
            pltpu.VMEM((_CHUNK, 2 * DIM), jnp.float32),
            pltpu.VMEM((_CHUNK, 2 * DIM), jnp.float32),
            pltpu.SemaphoreType.DMA,
        ],
    )
    def k(t_hbm, i0_hbm, i1_hbm, o0, o1, i0_v, i1_v, b0, b1, sem):
        wid = lax.axis_index("s") * _NC + lax.axis_index("c")
        base = wid * _BPW
        pltpu.sync_copy(i0_hbm.at[pl.ds(base, _BPW)], i0_v)
        pltpu.sync_copy(i1_hbm.at[pl.ds(base, _BPW)], i1_v)

        def body(c, _):
            off = c * _CHUNK
            cp0 = pltpu.async_copy(t_hbm.at[i0_v.at[pl.ds(off, _CHUNK)]], b0, sem)
            cp1 = pltpu.async_copy(t_hbm.at[i1_v.at[pl.ds(off, _CHUNK)]], b1, sem)
            cp0.wait()
            cp1.wait()
            dst = pl.ds(base + off, _CHUNK)
            pltpu.sync_copy(b0, o0.at[dst])
            pltpu.sync_copy(b1, o1.at[dst])

        lax.fori_loop(0, _NCHUNK, body, None, unroll=False)

    return k(table, idx0, idx1)


def _sc_gather1(table, idx0, width):
    mesh = plsc.VectorSubcoreMesh(core_axis_name="c", subcore_axis_name="s")
    out_t = jax.ShapeDtypeStruct((B, width), jnp.float32)

    @functools.partial(
        pl.kernel,
        mesh=mesh,
        out_type=out_t,
        scratch_types=[
            pltpu.VMEM((_BPW,), jnp.int32),
            pltpu.VMEM((_CHUNK, width), jnp.float32),
            pltpu.SemaphoreType.DMA,
        ],
    )
    def k(t_hbm, i0_hbm, o0, i0_v, b0, sem):
        wid = lax.axis_index("s") * _NC + lax.axis_index("c")
        base = wid * _BPW
        pltpu.sync_copy(i0_hbm.at[pl.ds(base, _BPW)], i0_v)

        def body(c, _):
            off = c * _CHUNK
            cp0 = pltpu.async_copy(t_hbm.at[i0_v.at[pl.ds(off, _CHUNK)]], b0, sem)
            cp0.wait()
            pltpu.sync_copy(b0, o0.at[pl.ds(base + off, _CHUNK)])

        lax.fori_loop(0, _NCHUNK, body, None, unroll=False)

    return k(table, idx0)


def _score_body(h_ref, t_ref, rw_ref, o_ref):
    blk = h_ref.shape[0]
    hh = h_ref[...]
    tt = t_ref[...]
    rw = rw_ref[...]
    lane = lax.broadcasted_iota(jnp.int32, (blk, 2 * DIM), 1)
    re_half = lane < DIM
    rw_swap = pltpu.roll(rw, DIM, 1)
    w2 = jnp.where(re_half, rw_swap, rw)
    cs = jnp.where(re_half, jnp.cos(rw),
                   pltpu.roll(jnp.sin(rw), DIM, 1))

    row = lax.broadcasted_iota(jnp.int32, (2 * DIM, 2 * DIM), 0)
    col = lax.broadcasted_iota(jnp.int32, (2 * DIM, 2 * DIM), 1)
    m = ((row < DIM) == (col < DIM)).astype(jnp.float32)

    def mxu(x, y):
        return jax.lax.dot(x, y, preferred_element_type=jnp.float32)

    d_h = mxu(w2 * hh, m)
    d_t = mxu(w2 * tt, m)
    p = hh - d_h * w2
    pt = tt - d_t * w2

    swap_cs = pltpu.roll(cs, DIM, 1)
    cc = jnp.where(re_half, cs, swap_cs)
    ssn = jnp.where(re_half, -swap_cs, cs)
    swap_p = pltpu.roll(p, DIM, 1)
    s = p * cc + swap_p * ssn - pt
    sq = s * s
    mag = jnp.sqrt(sq + pltpu.roll(sq, DIM, 1))
    ones = jnp.ones((2 * DIM, 1), jnp.float32)
    o_ref[...] = 0.5 * mxu(mag, ones) - GAMMA


def _tc_score(h, t, rw):
    blk = 2048
    return pl.pallas_call(
        _score_body,
        grid=(B // blk,),
        in_specs=[
            pl.BlockSpec((blk, 2 * DIM), lambda i: (i, 0)),
            pl.BlockSpec((blk, 2 * DIM), lambda i: (i, 0)),
            pl.BlockSpec((blk, 2 * DIM), lambda i: (i, 0)),
        ],
        out_specs=pl.BlockSpec((blk, 1), lambda i: (i, 0)),
        out_shape=jax.ShapeDtypeStruct((B, 1), jnp.float32),
    )(h, t, rw)


def kernel(pos_sample, ent_embd, ent_embd_im, rel_embd, wr):
    h_idx = pos_sample[:, 0]
    r_idx = pos_sample[:, 1]
    t_idx = pos_sample[:, 2]
    entcat = _entcat(ent_embd.T, ent_embd_im.T)
    h, t = _sc_gather2(entcat, h_idx, t_idx)
    relcat = _entcat(rel_embd.T, wr.T)
    r = _sc_gather1(relcat, r_idx, 2 * DIM)
    return _tc_score(h, t, r)
